# baseline (speedup 1.0000x reference)
import functools
import math

import jax
import jax.numpy as jnp
from jax import lax
from jax.experimental import pallas as pl
from jax.experimental.pallas import tpu as pltpu
from jax.experimental.pallas import tpu_sc as plsc

_VOCAB = 1000000
_EMB = 64
_B = 4096
_L = 200

_NC = 2
_NS = 16
_NW = _NC * _NS
_K = 128
_RPW = (_B * _L) // _NW
_NCHUNK = _RPW // _K
_SCALE = math.sqrt(_EMB)

_mesh = plsc.VectorSubcoreMesh(core_axis_name="c", subcore_axis_name="s")


@functools.partial(
    pl.kernel,
    mesh=_mesh,
    out_type=jax.ShapeDtypeStruct((_NW * _NCHUNK, _K, _EMB), jnp.float32),
    scratch_types=[
        pltpu.VMEM((_NCHUNK, _K), jnp.int32),
        pltpu.VMEM((_K, _EMB), jnp.float32),
        pltpu.SemaphoreType.DMA,
    ],
    compiler_params=pltpu.CompilerParams(use_tc_tiling_on_sc=False),
)
def _embed(tok_hbm, table_hbm, out_hbm, idx_v, rows_v, sem):
    wid = lax.axis_index("s") * _NC + lax.axis_index("c")
    pltpu.sync_copy(tok_hbm.at[wid], idx_v)

    def chunk(j, carry):
        pltpu.async_copy(table_hbm.at[idx_v.at[j]], rows_v, sem).wait()

        def scale_row(r, c2):
            for c in range(_EMB // 16):
                sl = (r, pl.ds(c * 16, 16))
                rows_v[sl] = rows_v[sl] * _SCALE
            return c2

        lax.fori_loop(0, _K, scale_row, None)
        pltpu.sync_copy(rows_v, out_hbm.at[wid * _NCHUNK + j])
        return carry

    lax.fori_loop(0, _NCHUNK, chunk, None)


def kernel(tokens, table):
    tok = tokens.reshape(_NW, _NCHUNK, _K).astype(jnp.int32)
    out = _embed(tok, table)
    return out.reshape(_B, _L, _EMB)


# speedup vs baseline: 1.2056x; 1.2056x over previous
import functools
import math

import jax
import jax.numpy as jnp
from jax import lax
from jax.experimental import pallas as pl
from jax.experimental.pallas import tpu as pltpu
from jax.experimental.pallas import tpu_sc as plsc

_VOCAB = 1000000
_EMB = 64
_B = 4096
_L = 200

_NC = 2
_NS = 16
_NW = _NC * _NS
_K = 128
_RPW = (_B * _L) // _NW
_NCHUNK = _RPW // _K
_NBUF = 4
_SCALE = math.sqrt(_EMB)
_RUNROLL = 8

_mesh = plsc.VectorSubcoreMesh(core_axis_name="c", subcore_axis_name="s")


@functools.partial(
    pl.kernel,
    mesh=_mesh,
    out_type=jax.ShapeDtypeStruct((_NW * _NCHUNK, _K, _EMB), jnp.float32),
    scratch_types=(
        [pltpu.VMEM((_NCHUNK, _K), jnp.int32)]
        + [pltpu.VMEM((_K, _EMB), jnp.float32) for _ in range(2 * _NBUF)]
        + [pltpu.SemaphoreType.DMA for _ in range(2 * _NBUF)]
    ),
    compiler_params=pltpu.CompilerParams(use_tc_tiling_on_sc=False),
)
def _embed(tok_hbm, table_hbm, out_hbm, idx_v, *bufs_and_sems):
    inb = bufs_and_sems[:_NBUF]
    oub = bufs_and_sems[_NBUF:2 * _NBUF]
    gsem = bufs_and_sems[2 * _NBUF:3 * _NBUF]
    ssem = bufs_and_sems[3 * _NBUF:4 * _NBUF]

    wid = lax.axis_index("s") * _NC + lax.axis_index("c")
    pltpu.sync_copy(tok_hbm.at[wid], idx_v)

    for b in range(_NBUF):
        pltpu.async_copy(table_hbm.at[idx_v.at[b]], inb[b], gsem[b])

    def scale_chunk(dst, src):
        def step(t, carry):
            for rr in range(_RUNROLL):
                r = t * _RUNROLL + rr
                for c in range(_EMB // 16):
                    sl = (r, pl.ds(c * 16, 16))
                    dst[sl] = src[sl] * _SCALE
            return carry

        lax.fori_loop(0, _K // _RUNROLL, step, None)

    def outer(it, carry):
        j0 = it * _NBUF
        for b in range(_NBUF):
            j = j0 + b
            pltpu.make_async_copy(
                table_hbm.at[idx_v.at[j]], inb[b], gsem[b]).wait()

            @pl.when(j0 > 0)
            def _():
                pltpu.make_async_copy(
                    oub[b], out_hbm.at[wid * _NCHUNK + j], ssem[b]).wait()

            scale_chunk(oub[b], inb[b])
            pltpu.async_copy(oub[b], out_hbm.at[wid * _NCHUNK + j], ssem[b])

            @pl.when(j0 + _NBUF < _NCHUNK)
            def _():
                pltpu.async_copy(
                    table_hbm.at[idx_v.at[j + _NBUF]], inb[b], gsem[b])

        return carry

    lax.fori_loop(0, _NCHUNK // _NBUF, outer, None)

    for b in range(_NBUF):
        j = _NCHUNK - _NBUF + b
        pltpu.make_async_copy(
            oub[b], out_hbm.at[wid * _NCHUNK + j], ssem[b]).wait()


def kernel(tokens, table):
    tok = tokens.reshape(_NW, _NCHUNK, _K).astype(jnp.int32)
    out = _embed(tok, table)
    return out.reshape(_B, _L, _EMB)
